# Initial kernel scaffold; baseline (speedup 1.0000x reference)
import jax
import jax.numpy as jnp
from jax.experimental import pallas as pl


def kernel(x, rows, cols, adj_weights, W_in, b_in, W_out, b_out):
    raise NotImplementedError("write your pallas kernel here")


# speedup vs baseline: 75.6901x; 75.6901x over previous
import functools

import jax
import jax.numpy as jnp
from jax import lax
from jax.experimental import pallas as pl
from jax.experimental.pallas import tpu as pltpu
from jax.experimental.pallas import tpu_sc as plsc

N = 65536
NNZ = 4194304
IN_DIM = 512
OUT_DIM = 512

NC = 2
NS = 16
NW = NC * NS
EPT = NNZ // NW
CH = 2048
NCHUNK = EPT // CH
L = 16


def _mv_in_body(w_ref, x_ref, b_ref, o_ref):
    acc = jnp.dot(w_ref[...], x_ref[...], preferred_element_type=jnp.float32)
    o_ref[...] = jnp.maximum(acc + b_ref[...], 0.0)


def _h_matvec(W_in, x, b_in):
    grid = N // 512
    return pl.pallas_call(
        _mv_in_body,
        grid=(grid,),
        in_specs=[
            pl.BlockSpec((512, IN_DIM), lambda i: (i, 0)),
            pl.BlockSpec((IN_DIM, 1), lambda i: (0, 0)),
            pl.BlockSpec((512, 1), lambda i: (i, 0)),
        ],
        out_specs=pl.BlockSpec((512, 1), lambda i: (i, 0)),
        out_shape=jax.ShapeDtypeStruct((N, 1), jnp.float32),
    )(W_in, x.reshape(IN_DIM, 1), b_in.reshape(N, 1))


def _mv_out_body(w_ref, yp_ref, b_ref, o_ref):
    i = pl.program_id(0)
    v = jnp.maximum(jnp.sum(yp_ref[...], axis=0), 0.0).reshape(512, 1)
    part = jnp.dot(w_ref[...], v, preferred_element_type=jnp.float32)

    @pl.when(i == 0)
    def _():
        o_ref[...] = b_ref[...] + part

    @pl.when(i > 0)
    def _():
        o_ref[...] += part


def _out_matvec(W_out, y_parts, b_out):
    grid = N // 512
    return pl.pallas_call(
        _mv_out_body,
        grid=(grid,),
        in_specs=[
            pl.BlockSpec((OUT_DIM, 512), lambda i: (0, i)),
            pl.BlockSpec((NW, 512), lambda i: (0, i)),
            pl.BlockSpec((OUT_DIM, 1), lambda i: (0, 0)),
        ],
        out_specs=pl.BlockSpec((OUT_DIM, 1), lambda i: (0, 0)),
        out_shape=jax.ShapeDtypeStruct((OUT_DIM, 1), jnp.float32),
    )(W_out, y_parts, b_out.reshape(OUT_DIM, 1))


def _sc_edge_body(h_hbm, cols_hbm, w_hbm, rows_hbm, yp_hbm, c_hbm,
                  hy_v, idx_v, val_v, c_v):
    wid = lax.axis_index("s") * NC + lax.axis_index("c")
    base = wid * EPT

    pltpu.sync_copy(h_hbm, hy_v)

    def chunk_a(ci, _):
        off = base + ci * CH
        pltpu.sync_copy(cols_hbm.at[pl.ds(off, CH)], idx_v)
        pltpu.sync_copy(w_hbm.at[pl.ds(off, CH)], val_v)

        def vec_a(j, _):
            s = pl.ds(j * L, L)
            g = plsc.load_gather(hy_v, [idx_v[s]])
            c_v[s] = g * val_v[s]
            return 0

        lax.fori_loop(0, CH // L, vec_a, 0)
        pltpu.sync_copy(c_v, c_hbm.at[pl.ds(off, CH)])
        return 0

    lax.fori_loop(0, NCHUNK, chunk_a, 0)

    zeros = jnp.zeros((L,), jnp.float32)

    def zero_body(i, _):
        hy_v[pl.ds(i * L, L)] = zeros
        return 0

    lax.fori_loop(0, N // L, zero_body, 0)

    def chunk_b(ci, _):
        off = base + ci * CH
        pltpu.sync_copy(rows_hbm.at[pl.ds(off, CH)], idx_v)
        pltpu.sync_copy(c_hbm.at[pl.ds(off, CH)], val_v)

        def vec_b(j, _):
            s = pl.ds(j * L, L)
            plsc.addupdate_scatter(hy_v, [idx_v[s]], val_v[s])
            return 0

        lax.fori_loop(0, CH // L, vec_b, 0)
        return 0

    lax.fori_loop(0, NCHUNK, chunk_b, 0)
    pltpu.sync_copy(hy_v, yp_hbm.at[wid])


_sc_edges = functools.partial(
    pl.kernel,
    out_type=(
        jax.ShapeDtypeStruct((NW, N), jnp.float32),
        jax.ShapeDtypeStruct((NNZ,), jnp.float32),
    ),
    mesh=plsc.VectorSubcoreMesh(
        core_axis_name="c", subcore_axis_name="s",
        num_cores=NC, num_subcores=NS,
    ),
    scratch_types=[
        pltpu.VMEM((N,), jnp.float32),
        pltpu.VMEM((CH,), jnp.int32),
        pltpu.VMEM((CH,), jnp.float32),
        pltpu.VMEM((CH,), jnp.float32),
    ],
    compiler_params=pltpu.CompilerParams(needs_layout_passes=False),
)(_sc_edge_body)


def kernel(x, rows, cols, adj_weights, W_in, b_in, W_out, b_out):
    h = _h_matvec(W_in, x, b_in)
    y_parts, _ = _sc_edges(h.reshape(N), cols, adj_weights, rows)
    out = _out_matvec(W_out, y_parts, b_out)
    return out.reshape(OUT_DIM)


# speedup vs baseline: 120.9079x; 1.5974x over previous
import functools

import jax
import jax.numpy as jnp
from jax import lax
from jax.experimental import pallas as pl
from jax.experimental.pallas import tpu as pltpu
from jax.experimental.pallas import tpu_sc as plsc

N = 65536
NNZ = 4194304
IN_DIM = 512
OUT_DIM = 512

NC = 2
NS = 16
NW = NC * NS
EPT = NNZ // NW
CH = 8192
NCHUNK = EPT // CH
NPAIR = NCHUNK // 2
L = 16


def _mv_in_body(w_ref, x_ref, b_ref, o_ref):
    acc = jnp.dot(w_ref[...], x_ref[...], preferred_element_type=jnp.float32)
    o_ref[...] = jnp.maximum(acc + b_ref[...], 0.0)


def _h_matvec(W_in, x, b_in):
    grid = N // 512
    return pl.pallas_call(
        _mv_in_body,
        grid=(grid,),
        in_specs=[
            pl.BlockSpec((512, IN_DIM), lambda i: (i, 0)),
            pl.BlockSpec((IN_DIM, 1), lambda i: (0, 0)),
            pl.BlockSpec((512, 1), lambda i: (i, 0)),
        ],
        out_specs=pl.BlockSpec((512, 1), lambda i: (i, 0)),
        out_shape=jax.ShapeDtypeStruct((N, 1), jnp.float32),
    )(W_in, x.reshape(IN_DIM, 1), b_in.reshape(N, 1))


def _mv_out_body(w_ref, yp_ref, b_ref, o_ref):
    i = pl.program_id(0)
    v = jnp.maximum(jnp.sum(yp_ref[...], axis=0), 0.0).reshape(512, 1)
    part = jnp.dot(w_ref[...], v, preferred_element_type=jnp.float32)

    @pl.when(i == 0)
    def _():
        o_ref[...] = b_ref[...] + part

    @pl.when(i > 0)
    def _():
        o_ref[...] += part


def _out_matvec(W_out, y_parts, b_out):
    grid = N // 512
    return pl.pallas_call(
        _mv_out_body,
        grid=(grid,),
        in_specs=[
            pl.BlockSpec((OUT_DIM, 512), lambda i: (0, i)),
            pl.BlockSpec((NW, 512), lambda i: (0, i)),
            pl.BlockSpec((OUT_DIM, 1), lambda i: (0, 0)),
        ],
        out_specs=pl.BlockSpec((OUT_DIM, 1), lambda i: (0, 0)),
        out_shape=jax.ShapeDtypeStruct((OUT_DIM, 1), jnp.float32),
    )(W_out, y_parts, b_out.reshape(OUT_DIM, 1))


def _sc_edge_body(h_hbm, cols_hbm, w_hbm, rows_hbm, yp_hbm, c_hbm,
                  hy_v, ia_v, ib_v, va_v, vb_v, ca_v, cb_v,
                  sia, sib, sva, svb, sca, scb):
    wid = lax.axis_index("s") * NC + lax.axis_index("c")
    base = wid * EPT

    def start_in(src, ci, buf, sem):
        pltpu.async_copy(src.at[pl.ds(base + ci * CH, CH)], buf, sem)

    def wait_in(src, buf, sem):
        pltpu.make_async_copy(src.at[pl.ds(base, CH)], buf, sem).wait()

    def start_out(buf, ci, sem):
        pltpu.async_copy(buf, c_hbm.at[pl.ds(base + ci * CH, CH)], sem)

    def wait_out(buf, sem):
        pltpu.make_async_copy(buf, c_hbm.at[pl.ds(base, CH)], sem).wait()

    pltpu.sync_copy(h_hbm, hy_v)
    start_in(cols_hbm, 0, ia_v, sia)
    start_in(w_hbm, 0, va_v, sva)

    def compute_a(idx_v, w_v, c_v):
        @plsc.parallel_loop(0, CH // L, unroll=8)
        def _(j):
            s = pl.ds(j * L, L)
            c_v[s] = plsc.load_gather(hy_v, [idx_v[s]]) * w_v[s]

    def pair_a(p, _):
        even = 2 * p
        start_in(cols_hbm, even + 1, ib_v, sib)
        start_in(w_hbm, even + 1, vb_v, svb)
        wait_in(cols_hbm, ia_v, sia)
        wait_in(w_hbm, va_v, sva)

        @pl.when(p > 0)
        def _():
            wait_out(ca_v, sca)

        compute_a(ia_v, va_v, ca_v)
        start_out(ca_v, even, sca)

        @pl.when(p < NPAIR - 1)
        def _():
            start_in(cols_hbm, even + 2, ia_v, sia)
            start_in(w_hbm, even + 2, va_v, sva)

        wait_in(cols_hbm, ib_v, sib)
        wait_in(w_hbm, vb_v, svb)

        @pl.when(p > 0)
        def _():
            wait_out(cb_v, scb)

        compute_a(ib_v, vb_v, cb_v)
        start_out(cb_v, even + 1, scb)
        return 0

    lax.fori_loop(0, NPAIR, pair_a, 0)
    wait_out(ca_v, sca)
    wait_out(cb_v, scb)

    zeros = jnp.zeros((L,), jnp.float32)

    @plsc.parallel_loop(0, N // L, unroll=8)
    def _(i):
        hy_v[pl.ds(i * L, L)] = zeros

    start_in(rows_hbm, 0, ia_v, sia)
    start_in(c_hbm, 0, va_v, sva)

    def compute_b(idx_v, c_v):
        def vec_b(j, _):
            s = pl.ds(j * L, L)
            plsc.addupdate_scatter(hy_v, [idx_v[s]], c_v[s])
            return 0

        lax.fori_loop(0, CH // L, vec_b, 0, unroll=8)

    def pair_b(p, _):
        even = 2 * p
        start_in(rows_hbm, even + 1, ib_v, sib)
        start_in(c_hbm, even + 1, vb_v, svb)
        wait_in(rows_hbm, ia_v, sia)
        wait_in(c_hbm, va_v, sva)
        compute_b(ia_v, va_v)

        @pl.when(p < NPAIR - 1)
        def _():
            start_in(rows_hbm, even + 2, ia_v, sia)
            start_in(c_hbm, even + 2, va_v, sva)

        wait_in(rows_hbm, ib_v, sib)
        wait_in(c_hbm, vb_v, svb)
        compute_b(ib_v, vb_v)
        return 0

    lax.fori_loop(0, NPAIR, pair_b, 0)
    pltpu.sync_copy(hy_v, yp_hbm.at[wid])


_sc_edges = functools.partial(
    pl.kernel,
    out_type=(
        jax.ShapeDtypeStruct((NW, N), jnp.float32),
        jax.ShapeDtypeStruct((NNZ,), jnp.float32),
    ),
    mesh=plsc.VectorSubcoreMesh(
        core_axis_name="c", subcore_axis_name="s",
        num_cores=NC, num_subcores=NS,
    ),
    scratch_types=[
        pltpu.VMEM((N,), jnp.float32),
        pltpu.VMEM((CH,), jnp.int32),
        pltpu.VMEM((CH,), jnp.int32),
        pltpu.VMEM((CH,), jnp.float32),
        pltpu.VMEM((CH,), jnp.float32),
        pltpu.VMEM((CH,), jnp.float32),
        pltpu.VMEM((CH,), jnp.float32),
        pltpu.SemaphoreType.DMA,
        pltpu.SemaphoreType.DMA,
        pltpu.SemaphoreType.DMA,
        pltpu.SemaphoreType.DMA,
        pltpu.SemaphoreType.DMA,
        pltpu.SemaphoreType.DMA,
    ],
    compiler_params=pltpu.CompilerParams(needs_layout_passes=False),
)(_sc_edge_body)


def kernel(x, rows, cols, adj_weights, W_in, b_in, W_out, b_out):
    h = _h_matvec(W_in, x, b_in)
    y_parts, _ = _sc_edges(h.reshape(N), cols, adj_weights, rows)
    out = _out_matvec(W_out, y_parts, b_out)
    return out.reshape(OUT_DIM)


# speedup vs baseline: 182.3503x; 1.5082x over previous
import functools

import jax
import jax.numpy as jnp
from jax import lax
from jax.experimental import pallas as pl
from jax.experimental.pallas import tpu as pltpu
from jax.experimental.pallas import tpu_sc as plsc

N = 65536
NNZ = 4194304
IN_DIM = 512
OUT_DIM = 512

NC = 2
NS = 16
NW = NC * NS
EPT = NNZ // NW
CH = 8192
NCHUNK = EPT // CH
NPAIR = NCHUNK // 2
L = 16


def _mv_in_body(w_ref, x_ref, b_ref, o_ref):
    acc = jnp.dot(w_ref[...], x_ref[...], preferred_element_type=jnp.float32)
    o_ref[...] = jnp.maximum(acc + b_ref[...], 0.0)


BM = 4096


def _h_matvec(W_in, x, b_in):
    grid = N // BM
    return pl.pallas_call(
        _mv_in_body,
        grid=(grid,),
        in_specs=[
            pl.BlockSpec((BM, IN_DIM), lambda i: (i, 0)),
            pl.BlockSpec((IN_DIM, 1), lambda i: (0, 0)),
            pl.BlockSpec((BM, 1), lambda i: (i, 0)),
        ],
        out_specs=pl.BlockSpec((BM, 1), lambda i: (i, 0)),
        out_shape=jax.ShapeDtypeStruct((N, 1), jnp.float32),
    )(W_in, x.reshape(IN_DIM, 1), b_in.reshape(N, 1))


BK = 4096


def _mv_out_body(w_ref, yp_ref, b_ref, o_ref):
    i = pl.program_id(0)
    v = jnp.maximum(jnp.sum(yp_ref[...], axis=0), 0.0).reshape(BK, 1)
    part = jnp.dot(w_ref[...], v, preferred_element_type=jnp.float32)

    @pl.when(i == 0)
    def _():
        o_ref[...] = b_ref[...] + part

    @pl.when(i > 0)
    def _():
        o_ref[...] += part


def _out_matvec(W_out, y_parts, b_out):
    grid = N // BK
    return pl.pallas_call(
        _mv_out_body,
        grid=(grid,),
        in_specs=[
            pl.BlockSpec((OUT_DIM, BK), lambda i: (0, i)),
            pl.BlockSpec((NW, BK), lambda i: (0, i)),
            pl.BlockSpec((OUT_DIM, 1), lambda i: (0, 0)),
        ],
        out_specs=pl.BlockSpec((OUT_DIM, 1), lambda i: (0, 0)),
        out_shape=jax.ShapeDtypeStruct((OUT_DIM, 1), jnp.float32),
    )(W_out, y_parts, b_out.reshape(OUT_DIM, 1))


def _sc_edge_body(h_hbm, cols_hbm, w_hbm, rows_hbm, yp_hbm, c_hbm,
                  hy_v, ia_v, ib_v, va_v, vb_v, ca_v, cb_v,
                  sia, sib, sva, svb, sca, scb):
    wid = lax.axis_index("s") * NC + lax.axis_index("c")
    base = wid * EPT

    def start_in(src, ci, buf, sem):
        pltpu.async_copy(src.at[pl.ds(base + ci * CH, CH)], buf, sem)

    def wait_in(src, buf, sem):
        pltpu.make_async_copy(src.at[pl.ds(base, CH)], buf, sem).wait()

    def start_out(buf, ci, sem):
        pltpu.async_copy(buf, c_hbm.at[pl.ds(base + ci * CH, CH)], sem)

    def wait_out(buf, sem):
        pltpu.make_async_copy(buf, c_hbm.at[pl.ds(base, CH)], sem).wait()

    pltpu.sync_copy(h_hbm, hy_v)
    start_in(cols_hbm, 0, ia_v, sia)
    start_in(w_hbm, 0, va_v, sva)

    def compute_a(idx_v, w_v, c_v):
        @plsc.parallel_loop(0, CH // L, unroll=8)
        def _(j):
            s = pl.ds(j * L, L)
            c_v[s] = plsc.load_gather(hy_v, [idx_v[s]]) * w_v[s]

    def pair_a(p, _):
        even = 2 * p
        start_in(cols_hbm, even + 1, ib_v, sib)
        start_in(w_hbm, even + 1, vb_v, svb)
        wait_in(cols_hbm, ia_v, sia)
        wait_in(w_hbm, va_v, sva)

        @pl.when(p > 0)
        def _():
            wait_out(ca_v, sca)

        compute_a(ia_v, va_v, ca_v)
        start_out(ca_v, even, sca)

        @pl.when(p < NPAIR - 1)
        def _():
            start_in(cols_hbm, even + 2, ia_v, sia)
            start_in(w_hbm, even + 2, va_v, sva)

        wait_in(cols_hbm, ib_v, sib)
        wait_in(w_hbm, vb_v, svb)

        @pl.when(p > 0)
        def _():
            wait_out(cb_v, scb)

        compute_a(ib_v, vb_v, cb_v)
        start_out(cb_v, even + 1, scb)
        return 0

    lax.fori_loop(0, NPAIR, pair_a, 0)
    wait_out(ca_v, sca)
    wait_out(cb_v, scb)

    zeros = jnp.zeros((L,), jnp.float32)

    @plsc.parallel_loop(0, N // L, unroll=8)
    def _(i):
        hy_v[pl.ds(i * L, L)] = zeros

    start_in(rows_hbm, 0, ia_v, sia)
    start_in(c_hbm, 0, va_v, sva)

    def compute_b(idx_v, c_v):
        def vec_b(j, _):
            s = pl.ds(j * L, L)
            plsc.addupdate_scatter(hy_v, [idx_v[s]], c_v[s])
            return 0

        lax.fori_loop(0, CH // L, vec_b, 0, unroll=8)

    def pair_b(p, _):
        even = 2 * p
        start_in(rows_hbm, even + 1, ib_v, sib)
        start_in(c_hbm, even + 1, vb_v, svb)
        wait_in(rows_hbm, ia_v, sia)
        wait_in(c_hbm, va_v, sva)
        compute_b(ia_v, va_v)

        @pl.when(p < NPAIR - 1)
        def _():
            start_in(rows_hbm, even + 2, ia_v, sia)
            start_in(c_hbm, even + 2, va_v, sva)

        wait_in(rows_hbm, ib_v, sib)
        wait_in(c_hbm, vb_v, svb)
        compute_b(ib_v, vb_v)
        return 0

    lax.fori_loop(0, NPAIR, pair_b, 0)
    pltpu.sync_copy(hy_v, yp_hbm.at[wid])


_sc_edges = functools.partial(
    pl.kernel,
    out_type=(
        jax.ShapeDtypeStruct((NW, N), jnp.float32),
        jax.ShapeDtypeStruct((NNZ,), jnp.float32),
    ),
    mesh=plsc.VectorSubcoreMesh(
        core_axis_name="c", subcore_axis_name="s",
        num_cores=NC, num_subcores=NS,
    ),
    scratch_types=[
        pltpu.VMEM((N,), jnp.float32),
        pltpu.VMEM((CH,), jnp.int32),
        pltpu.VMEM((CH,), jnp.int32),
        pltpu.VMEM((CH,), jnp.float32),
        pltpu.VMEM((CH,), jnp.float32),
        pltpu.VMEM((CH,), jnp.float32),
        pltpu.VMEM((CH,), jnp.float32),
        pltpu.SemaphoreType.DMA,
        pltpu.SemaphoreType.DMA,
        pltpu.SemaphoreType.DMA,
        pltpu.SemaphoreType.DMA,
        pltpu.SemaphoreType.DMA,
        pltpu.SemaphoreType.DMA,
    ],
    compiler_params=pltpu.CompilerParams(needs_layout_passes=False),
)(_sc_edge_body)


def kernel(x, rows, cols, adj_weights, W_in, b_in, W_out, b_out):
    h = _h_matvec(W_in, x, b_in)
    y_parts, _ = _sc_edges(h.reshape(N), cols, adj_weights, rows)
    out = _out_matvec(W_out, y_parts, b_out)
    return out.reshape(OUT_DIM)
